# Initial kernel scaffold; baseline (speedup 1.0000x reference)
import jax
import jax.numpy as jnp
from jax.experimental import pallas as pl


def kernel(x, edge_index, edge_weight, W1, b1, W2, b2, Wp1, bp1, Wp2, bp2):
    raise NotImplementedError("write your pallas kernel here")


# speedup vs baseline: 3.4144x; 3.4144x over previous
import functools

import jax
import jax.numpy as jnp
from jax import lax
from jax.experimental import pallas as pl
from jax.experimental.pallas import tpu as pltpu
from jax.experimental.pallas import tpu_sc as plsc

N = 10000
E = 320000
D = 128
NC = 2
NS = 16
LANES = 16
EPT = 10240
CHUNK = 128
NCH = EPT // CHUNK
E_PAD = NC * NS * EPT
ROWS_PER_SUB = 624
ROWS_TAIL = N - NS * ROWS_PER_SUB


def _spmm_sc(table, src3, dst3, w3, zeros):
    mesh = plsc.VectorSubcoreMesh(core_axis_name="c", subcore_axis_name="s")

    @functools.partial(
        pl.kernel,
        out_type=jax.ShapeDtypeStruct((NC, N, D), jnp.float32),
        mesh=mesh,
        scratch_types=[
            pltpu.VMEM_SHARED((N, D), jnp.float32),
            pltpu.VMEM((NCH, CHUNK), jnp.int32),
            pltpu.VMEM((NCH, CHUNK), jnp.int32),
            pltpu.VMEM((NCH, CHUNK), jnp.float32),
            pltpu.VMEM((CHUNK, D), jnp.float32),
            pltpu.SemaphoreType.DMA,
        ],
    )
    def spmm_kernel(table_hbm, src_hbm, dst_hbm, w_hbm, zeros_hbm, out_hbm,
                    acc, srcb, dstb, wb, rows, sem):
        c = lax.axis_index("c")
        s = lax.axis_index("s")
        tile = c * NS + s

        pltpu.sync_copy(src_hbm.at[tile], srcb)
        pltpu.sync_copy(dst_hbm.at[tile], dstb)
        pltpu.sync_copy(w_hbm.at[tile], wb)

        rbase = pl.multiple_of(s * ROWS_PER_SUB, 8)
        pltpu.sync_copy(zeros_hbm.at[pl.ds(rbase, ROWS_PER_SUB)],
                        acc.at[pl.ds(rbase, ROWS_PER_SUB)])

        @pl.when(s == NS - 1)
        def _zero_tail():
            pltpu.sync_copy(zeros_hbm.at[pl.ds(NS * ROWS_PER_SUB, ROWS_TAIL)],
                            acc.at[pl.ds(NS * ROWS_PER_SUB, ROWS_TAIL)])

        plsc.subcore_barrier()

        def chunk_body(j, _):
            pltpu.async_copy(table_hbm.at[srcb.at[j]], rows, sem).wait()

            def group_body(g, _):
                wvec = wb[j, pl.ds(g * LANES, LANES)]
                for l in range(LANES):
                    ws = wvec[l]
                    k = g * LANES + l
                    for jj in range(D // LANES):
                        sl = pl.ds(jj * LANES, LANES)
                        rows[k, sl] = rows[k, sl] * ws
                return 0

            lax.fori_loop(0, CHUNK // LANES, group_body, 0)

            pltpu.sync_copy(rows, acc.at[dstb.at[j]], add=True)
            return 0

        lax.fori_loop(0, NCH, chunk_body, 0)

        plsc.subcore_barrier()
        pltpu.sync_copy(acc.at[pl.ds(rbase, ROWS_PER_SUB)],
                        out_hbm.at[c, pl.ds(rbase, ROWS_PER_SUB)])

        @pl.when(s == NS - 1)
        def _write_tail():
            pltpu.sync_copy(acc.at[pl.ds(NS * ROWS_PER_SUB, ROWS_TAIL)],
                            out_hbm.at[c, pl.ds(NS * ROWS_PER_SUB, ROWS_TAIL)])

    return spmm_kernel(table, src3, dst3, w3, zeros)


_RB = 1000

def _mlp_body(p_ref, w1_ref, b1_ref, w2_ref, out_ref):
    agg = p_ref[0] + p_ref[1]
    h = jnp.dot(agg, w1_ref[...], preferred_element_type=jnp.float32,
                precision=lax.Precision.HIGHEST) + b1_ref[...]
    h = jnp.maximum(h, 0.0)
    out_ref[...] = jnp.dot(h, w2_ref[...], preferred_element_type=jnp.float32,
                           precision=lax.Precision.HIGHEST)


def _mlp_tc(p1, W1, b1, W2):
    grid = N // _RB
    return pl.pallas_call(
        _mlp_body,
        grid=(grid,),
        in_specs=[
            pl.BlockSpec((NC, _RB, D), lambda i: (0, i, 0)),
            pl.BlockSpec((D, 256), lambda i: (0, 0)),
            pl.BlockSpec((1, 256), lambda i: (0, 0)),
            pl.BlockSpec((256, D), lambda i: (0, 0)),
        ],
        out_specs=pl.BlockSpec((_RB, D), lambda i: (i, 0)),
        out_shape=jax.ShapeDtypeStruct((N, D), jnp.float32),
    )(p1, W1, b1.reshape(1, 256), W2)


def _proj_body(p_ref, b2_ref, wp1_ref, bp1_ref, wp2_ref, bp2_ref,
               z_ref, emb_ref):
    emb = p_ref[0] + p_ref[1] + b2_ref[...]
    emb_ref[...] = emb
    t = jnp.dot(emb, wp1_ref[...], preferred_element_type=jnp.float32,
                precision=lax.Precision.HIGHEST) + bp1_ref[...]
    t = jnp.maximum(t, 0.0)
    z_ref[...] = jnp.dot(t, wp2_ref[...], preferred_element_type=jnp.float32,
                         precision=lax.Precision.HIGHEST) + bp2_ref[...]


def _proj_tc(p2, b2, Wp1, bp1, Wp2, bp2):
    grid = N // _RB
    return pl.pallas_call(
        _proj_body,
        grid=(grid,),
        in_specs=[
            pl.BlockSpec((NC, _RB, D), lambda i: (0, i, 0)),
            pl.BlockSpec((1, D), lambda i: (0, 0)),
            pl.BlockSpec((D, D), lambda i: (0, 0)),
            pl.BlockSpec((1, D), lambda i: (0, 0)),
            pl.BlockSpec((D, D), lambda i: (0, 0)),
            pl.BlockSpec((1, D), lambda i: (0, 0)),
        ],
        out_specs=[
            pl.BlockSpec((_RB, D), lambda i: (i, 0)),
            pl.BlockSpec((_RB, D), lambda i: (i, 0)),
        ],
        out_shape=[
            jax.ShapeDtypeStruct((N, D), jnp.float32),
            jax.ShapeDtypeStruct((N, D), jnp.float32),
        ],
    )(p2, b2.reshape(1, D), Wp1, bp1.reshape(1, D), Wp2, bp2.reshape(1, D))


def kernel(x, edge_index, edge_weight, W1, b1, W2, b2, Wp1, bp1, Wp2, bp2):
    pad = E_PAD - E
    src3 = jnp.concatenate(
        [edge_index[0], jnp.zeros((pad,), jnp.int32)]).reshape(NC * NS, NCH, CHUNK)
    dst3 = jnp.concatenate(
        [edge_index[1], jnp.zeros((pad,), jnp.int32)]).reshape(NC * NS, NCH, CHUNK)
    w3 = jnp.concatenate(
        [edge_weight, jnp.zeros((pad,), jnp.float32)]).reshape(NC * NS, NCH, CHUNK)
    zeros = jnp.zeros((N, D), jnp.float32)

    p1 = _spmm_sc(x, src3, dst3, w3, zeros)
    s2 = _mlp_tc(p1, W1, b1, W2)
    p2 = _spmm_sc(s2, src3, dst3, w3, zeros)
    z, emb = _proj_tc(p2, b2, Wp1, bp1, Wp2, bp2)
    return (z, emb)


# speedup vs baseline: 3.9406x; 1.1541x over previous
import functools

import jax
import jax.numpy as jnp
from jax import lax
from jax.experimental import pallas as pl
from jax.experimental.pallas import tpu as pltpu
from jax.experimental.pallas import tpu_sc as plsc

N = 10000
E = 320000
D = 128
NC = 2
NS = 16
LANES = 16
EPT = 10240
CHUNK = 128
NCH = EPT // CHUNK
E_PAD = NC * NS * EPT
ROWS_PER_SUB = 624
ROWS_TAIL = N - NS * ROWS_PER_SUB


def _spmm_sc(table, edata, w3, zeros):
    mesh = plsc.VectorSubcoreMesh(core_axis_name="c", subcore_axis_name="s")

    @functools.partial(
        pl.kernel,
        out_type=jax.ShapeDtypeStruct((NC, N, D), jnp.float32),
        mesh=mesh,
        scratch_types=[
            pltpu.VMEM_SHARED((N, D), jnp.float32),
            pltpu.VMEM((NCH, CHUNK), jnp.float32),
            pltpu.VMEM((2, CHUNK), jnp.int32),
            pltpu.VMEM((2, CHUNK), jnp.int32),
            pltpu.VMEM((CHUNK, D), jnp.float32),
            pltpu.VMEM((CHUNK, D), jnp.float32),
            pltpu.SemaphoreType.DMA,
            pltpu.SemaphoreType.DMA,
            pltpu.SemaphoreType.DMA,
            pltpu.SemaphoreType.DMA,
        ],
    )
    def spmm_kernel(table_hbm, edata_hbm, w_hbm, zeros_hbm, out_hbm,
                    acc, wb, ebuf0, ebuf1, rows0, rows1,
                    gsem0, gsem1, esem0, esem1):
        c = lax.axis_index("c")
        s = lax.axis_index("s")
        tile = c * NS + s

        pltpu.sync_copy(w_hbm.at[tile], wb)

        rbase = pl.multiple_of(s * ROWS_PER_SUB, 8)
        pltpu.sync_copy(zeros_hbm.at[pl.ds(rbase, ROWS_PER_SUB)],
                        acc.at[pl.ds(rbase, ROWS_PER_SUB)])

        @pl.when(s == NS - 1)
        def _zero_tail():
            pltpu.sync_copy(zeros_hbm.at[pl.ds(NS * ROWS_PER_SUB, ROWS_TAIL)],
                            acc.at[pl.ds(NS * ROWS_PER_SUB, ROWS_TAIL)])

        plsc.subcore_barrier()

        bufs = ((rows0, gsem0, ebuf0, esem0), (rows1, gsem1, ebuf1, esem1))

        def process(j, rows, ebuf):
            def group_body(g, _):
                wvec = wb[j, pl.ds(g * LANES, LANES)]
                for l in range(LANES):
                    ws = wvec[l]
                    k = g * LANES + l
                    for jj in range(D // LANES):
                        sl = pl.ds(jj * LANES, LANES)
                        rows[k, sl] = rows[k, sl] * ws
                return 0

            lax.fori_loop(0, CHUNK // LANES, group_body, 0)

            pltpu.sync_copy(rows, acc.at[ebuf.at[1]], add=True)

        pltpu.async_copy(edata_hbm.at[tile, 0], ebuf0, esem0)
        pltpu.async_copy(edata_hbm.at[tile, 1], ebuf1, esem1)
        pltpu.make_async_copy(edata_hbm.at[tile, 0], ebuf0, esem0).wait()
        pltpu.async_copy(table_hbm.at[ebuf0.at[0]], rows0, gsem0)

        @pl.loop(0, NCH, step=2)
        def _pipe(jo):
            for b, (rows, gsem, ebuf, esem) in enumerate(bufs):
                j = jo + b
                rows_n, gsem_n, ebuf_n, esem_n = bufs[1 - b]

                pltpu.make_async_copy(table_hbm.at[ebuf.at[0]], rows, gsem).wait()

                @pl.when(j < NCH - 1)
                def _issue_next_gather():
                    pltpu.make_async_copy(
                        edata_hbm.at[tile, j + 1], ebuf_n, esem_n).wait()
                    pltpu.async_copy(table_hbm.at[ebuf_n.at[0]], rows_n, gsem_n)

                process(j, rows, ebuf)

                @pl.when(j < NCH - 2)
                def _issue_next_estage():
                    pltpu.async_copy(edata_hbm.at[tile, j + 2], ebuf, esem)

        plsc.subcore_barrier()
        pltpu.sync_copy(acc.at[pl.ds(rbase, ROWS_PER_SUB)],
                        out_hbm.at[c, pl.ds(rbase, ROWS_PER_SUB)])

        @pl.when(s == NS - 1)
        def _write_tail():
            pltpu.sync_copy(acc.at[pl.ds(NS * ROWS_PER_SUB, ROWS_TAIL)],
                            out_hbm.at[c, pl.ds(NS * ROWS_PER_SUB, ROWS_TAIL)])

    return spmm_kernel(table, edata, w3, zeros)


_RB = 1000

def _mlp_body(p_ref, w1_ref, b1_ref, w2_ref, out_ref):
    agg = p_ref[0] + p_ref[1]
    h = jnp.dot(agg, w1_ref[...], preferred_element_type=jnp.float32,
                precision=lax.Precision.HIGHEST) + b1_ref[...]
    h = jnp.maximum(h, 0.0)
    out_ref[...] = jnp.dot(h, w2_ref[...], preferred_element_type=jnp.float32,
                           precision=lax.Precision.HIGHEST)


def _mlp_tc(p1, W1, b1, W2):
    grid = N // _RB
    return pl.pallas_call(
        _mlp_body,
        grid=(grid,),
        in_specs=[
            pl.BlockSpec((NC, _RB, D), lambda i: (0, i, 0)),
            pl.BlockSpec((D, 256), lambda i: (0, 0)),
            pl.BlockSpec((1, 256), lambda i: (0, 0)),
            pl.BlockSpec((256, D), lambda i: (0, 0)),
        ],
        out_specs=pl.BlockSpec((_RB, D), lambda i: (i, 0)),
        out_shape=jax.ShapeDtypeStruct((N, D), jnp.float32),
    )(p1, W1, b1.reshape(1, 256), W2)


def _proj_body(p_ref, b2_ref, wp1_ref, bp1_ref, wp2_ref, bp2_ref,
               z_ref, emb_ref):
    emb = p_ref[0] + p_ref[1] + b2_ref[...]
    emb_ref[...] = emb
    t = jnp.dot(emb, wp1_ref[...], preferred_element_type=jnp.float32,
                precision=lax.Precision.HIGHEST) + bp1_ref[...]
    t = jnp.maximum(t, 0.0)
    z_ref[...] = jnp.dot(t, wp2_ref[...], preferred_element_type=jnp.float32,
                         precision=lax.Precision.HIGHEST) + bp2_ref[...]


def _proj_tc(p2, b2, Wp1, bp1, Wp2, bp2):
    grid = N // _RB
    return pl.pallas_call(
        _proj_body,
        grid=(grid,),
        in_specs=[
            pl.BlockSpec((NC, _RB, D), lambda i: (0, i, 0)),
            pl.BlockSpec((1, D), lambda i: (0, 0)),
            pl.BlockSpec((D, D), lambda i: (0, 0)),
            pl.BlockSpec((1, D), lambda i: (0, 0)),
            pl.BlockSpec((D, D), lambda i: (0, 0)),
            pl.BlockSpec((1, D), lambda i: (0, 0)),
        ],
        out_specs=[
            pl.BlockSpec((_RB, D), lambda i: (i, 0)),
            pl.BlockSpec((_RB, D), lambda i: (i, 0)),
        ],
        out_shape=[
            jax.ShapeDtypeStruct((N, D), jnp.float32),
            jax.ShapeDtypeStruct((N, D), jnp.float32),
        ],
    )(p2, b2.reshape(1, D), Wp1, bp1.reshape(1, D), Wp2, bp2.reshape(1, D))


def kernel(x, edge_index, edge_weight, W1, b1, W2, b2, Wp1, bp1, Wp2, bp2):
    pad = E_PAD - E
    src3 = jnp.concatenate(
        [edge_index[0], jnp.zeros((pad,), jnp.int32)]).reshape(NC * NS, NCH, CHUNK)
    dst3 = jnp.concatenate(
        [edge_index[1], jnp.zeros((pad,), jnp.int32)]).reshape(NC * NS, NCH, CHUNK)
    w3 = jnp.concatenate(
        [edge_weight, jnp.zeros((pad,), jnp.float32)]).reshape(NC * NS, NCH, CHUNK)
    edata = jnp.stack([src3, dst3], axis=2)
    zeros = jnp.zeros((N, D), jnp.float32)

    p1 = _spmm_sc(x, edata, w3, zeros)
    s2 = _mlp_tc(p1, W1, b1, W2)
    p2 = _spmm_sc(s2, edata, w3, zeros)
    z, emb = _proj_tc(p2, b2, Wp1, bp1, Wp2, bp2)
    return (z, emb)


# speedup vs baseline: 4.1685x; 1.0578x over previous
import functools

import jax
import jax.numpy as jnp
from jax import lax
from jax.experimental import pallas as pl
from jax.experimental.pallas import tpu as pltpu
from jax.experimental.pallas import tpu_sc as plsc

N = 10000
E = 320000
D = 128
DH = 64
NP = N // 2
NC = 2
NS = 16
LANES = 16
EPT = 20480
CHUNK = 64
NCH = EPT // CHUNK
E_PAD = NS * EPT
ROWS_PER_SUB = 312
ROWS_TAIL = NP - NS * ROWS_PER_SUB


def _spmm_sc(table2, edata, wdata, zeros):
    mesh = plsc.VectorSubcoreMesh(core_axis_name="c", subcore_axis_name="s")

    @functools.partial(
        pl.kernel,
        out_type=jax.ShapeDtypeStruct((NC, NP, D), jnp.float32),
        mesh=mesh,
        scratch_types=[
            pltpu.VMEM_SHARED((NP, D), jnp.float32),
            pltpu.VMEM_SHARED((NP, D), jnp.float32),
            pltpu.VMEM((4, CHUNK), jnp.float32),
            pltpu.VMEM((4, CHUNK), jnp.float32),
            pltpu.VMEM((2, CHUNK), jnp.int32),
            pltpu.VMEM((2, CHUNK), jnp.int32),
            pltpu.VMEM((CHUNK, D), jnp.float32),
            pltpu.VMEM((CHUNK, D), jnp.float32),
            pltpu.VMEM((CHUNK, D), jnp.float32),
            pltpu.SemaphoreType.DMA,
            pltpu.SemaphoreType.DMA,
            pltpu.SemaphoreType.DMA,
            pltpu.SemaphoreType.DMA,
            pltpu.SemaphoreType.DMA,
            pltpu.SemaphoreType.DMA,
        ],
    )
    def spmm_kernel(table_hbm, edata_hbm, w_hbm, zeros_hbm, out_hbm,
                    tbl, acc, wbuf0, wbuf1, ebuf0, ebuf1, rows0, rows1, sbuf,
                    gsem0, gsem1, esem0, esem1, wsem0, wsem1):
        c = lax.axis_index("c")
        s = lax.axis_index("s")

        rbase = pl.multiple_of(s * ROWS_PER_SUB, 8)
        pltpu.sync_copy(table_hbm.at[c, pl.ds(rbase, ROWS_PER_SUB)],
                        tbl.at[pl.ds(rbase, ROWS_PER_SUB)])
        pltpu.sync_copy(zeros_hbm.at[pl.ds(rbase, ROWS_PER_SUB)],
                        acc.at[pl.ds(rbase, ROWS_PER_SUB)])

        @pl.when(s == NS - 1)
        def _stage_tail():
            tsl = pl.ds(NS * ROWS_PER_SUB, ROWS_TAIL)
            pltpu.sync_copy(table_hbm.at[c, tsl], tbl.at[tsl])
            pltpu.sync_copy(zeros_hbm.at[tsl], acc.at[tsl])

        plsc.subcore_barrier()

        bufs = ((rows0, gsem0, ebuf0, esem0, wbuf0, wsem0),
                (rows1, gsem1, ebuf1, esem1, wbuf1, wsem1))

        def process(rows, ebuf, wbuf):
            def group_body(g, _):
                base = g * LANES
                av = wbuf[0, pl.ds(base, LANES)]
                bv = wbuf[1, pl.ds(base, LANES)]
                cv = wbuf[2, pl.ds(base, LANES)]
                dv = wbuf[3, pl.ds(base, LANES)]
                for l in range(LANES):
                    k = base + l
                    a, b, cc, dd = av[l], bv[l], cv[l], dv[l]
                    for jj in range(DH // LANES):
                        slo = pl.ds(jj * LANES, LANES)
                        shi = pl.ds(DH + jj * LANES, LANES)
                        lo = rows[k, slo]
                        hi = rows[k, shi]
                        sbuf[k, slo] = lo * a + hi * b
                        sbuf[k, shi] = lo * cc + hi * dd
                return 0

            lax.fori_loop(0, CHUNK // LANES, group_body, 0)

            pltpu.sync_copy(sbuf, acc.at[ebuf.at[1]], add=True)

        pltpu.async_copy(edata_hbm.at[s, 0], ebuf0, esem0)
        pltpu.async_copy(edata_hbm.at[s, 1], ebuf1, esem1)
        pltpu.async_copy(w_hbm.at[s, 0], wbuf0, wsem0)
        pltpu.async_copy(w_hbm.at[s, 1], wbuf1, wsem1)
        pltpu.make_async_copy(edata_hbm.at[s, 0], ebuf0, esem0).wait()
        pltpu.async_copy(tbl.at[ebuf0.at[0]], rows0, gsem0)

        @pl.loop(0, NCH, step=2)
        def _pipe(jo):
            for b, (rows, gsem, ebuf, esem, wbuf, wsem) in enumerate(bufs):
                j = jo + b
                rows_n, gsem_n, ebuf_n, esem_n, wbuf_n, wsem_n = bufs[1 - b]

                pltpu.make_async_copy(tbl.at[ebuf.at[0]], rows, gsem).wait()

                @pl.when(j < NCH - 1)
                def _issue_next_gather():
                    pltpu.make_async_copy(
                        edata_hbm.at[s, j + 1], ebuf_n, esem_n).wait()
                    pltpu.async_copy(tbl.at[ebuf_n.at[0]], rows_n, gsem_n)

                pltpu.make_async_copy(w_hbm.at[s, j], wbuf, wsem).wait()
                process(rows, ebuf, wbuf)

                @pl.when(j < NCH - 2)
                def _issue_next_estage():
                    pltpu.async_copy(edata_hbm.at[s, j + 2], ebuf, esem)
                    pltpu.async_copy(w_hbm.at[s, j + 2], wbuf, wsem)

        plsc.subcore_barrier()
        pltpu.sync_copy(acc.at[pl.ds(rbase, ROWS_PER_SUB)],
                        out_hbm.at[c, pl.ds(rbase, ROWS_PER_SUB)])

        @pl.when(s == NS - 1)
        def _write_tail():
            tsl = pl.ds(NS * ROWS_PER_SUB, ROWS_TAIL)
            pltpu.sync_copy(acc.at[tsl], out_hbm.at[c, tsl])

    return spmm_kernel(table2, edata, wdata, zeros)


_RB = 1000

def _mlp_body(p_ref, w1_ref, b1_ref, w2_ref, out_ref):
    agg = jnp.concatenate([p_ref[0], p_ref[1]], axis=-1)
    h = jnp.dot(agg, w1_ref[...], preferred_element_type=jnp.float32,
                precision=lax.Precision.HIGHEST) + b1_ref[...]
    h = jnp.maximum(h, 0.0)
    s2 = jnp.dot(h, w2_ref[...], preferred_element_type=jnp.float32,
                 precision=lax.Precision.HIGHEST)
    out_ref[0] = s2[:, :DH]
    out_ref[1] = s2[:, DH:]


def _mlp_tc(p1, W1, b1, W2):
    grid = N // _RB
    return pl.pallas_call(
        _mlp_body,
        grid=(grid,),
        in_specs=[
            pl.BlockSpec((NC, _RB, DH), lambda i: (0, i, 0)),
            pl.BlockSpec((D, 256), lambda i: (0, 0)),
            pl.BlockSpec((1, 256), lambda i: (0, 0)),
            pl.BlockSpec((256, D), lambda i: (0, 0)),
        ],
        out_specs=pl.BlockSpec((NC, _RB, DH), lambda i: (0, i, 0)),
        out_shape=jax.ShapeDtypeStruct((NC, N, DH), jnp.float32),
    )(p1, W1, b1.reshape(1, 256), W2)


def _proj_body(p_ref, b2_ref, wp1_ref, bp1_ref, wp2_ref, bp2_ref,
               z_ref, emb_ref):
    emb = jnp.concatenate([p_ref[0], p_ref[1]], axis=-1) + b2_ref[...]
    emb_ref[...] = emb
    t = jnp.dot(emb, wp1_ref[...], preferred_element_type=jnp.float32,
                precision=lax.Precision.HIGHEST) + bp1_ref[...]
    t = jnp.maximum(t, 0.0)
    z_ref[...] = jnp.dot(t, wp2_ref[...], preferred_element_type=jnp.float32,
                         precision=lax.Precision.HIGHEST) + bp2_ref[...]


def _proj_tc(p2, b2, Wp1, bp1, Wp2, bp2):
    grid = N // _RB
    return pl.pallas_call(
        _proj_body,
        grid=(grid,),
        in_specs=[
            pl.BlockSpec((NC, _RB, DH), lambda i: (0, i, 0)),
            pl.BlockSpec((1, D), lambda i: (0, 0)),
            pl.BlockSpec((D, D), lambda i: (0, 0)),
            pl.BlockSpec((1, D), lambda i: (0, 0)),
            pl.BlockSpec((D, D), lambda i: (0, 0)),
            pl.BlockSpec((1, D), lambda i: (0, 0)),
        ],
        out_specs=[
            pl.BlockSpec((_RB, D), lambda i: (i, 0)),
            pl.BlockSpec((_RB, D), lambda i: (i, 0)),
        ],
        out_shape=[
            jax.ShapeDtypeStruct((N, D), jnp.float32),
            jax.ShapeDtypeStruct((N, D), jnp.float32),
        ],
    )(p2, b2.reshape(1, D), Wp1, bp1.reshape(1, D), Wp2, bp2.reshape(1, D))


def kernel(x, edge_index, edge_weight, W1, b1, W2, b2, Wp1, bp1, Wp2, bp2):
    pad = E_PAD - E
    src = jnp.concatenate([edge_index[0], jnp.zeros((pad,), jnp.int32)])
    dst = jnp.concatenate([edge_index[1], jnp.zeros((pad,), jnp.int32)])
    w = jnp.concatenate([edge_weight, jnp.zeros((pad,), jnp.float32)])

    gidx = lax.shift_right_logical(src, 1).reshape(NS, NCH, CHUNK)
    didx = lax.shift_right_logical(dst, 1).reshape(NS, NCH, CHUNK)
    sp = (src & 1).astype(jnp.float32)
    dp = (dst & 1).astype(jnp.float32)
    a = (w * (1 - sp) * (1 - dp)).reshape(NS, NCH, 1, CHUNK)
    b = (w * sp * (1 - dp)).reshape(NS, NCH, 1, CHUNK)
    cf = (w * (1 - sp) * dp).reshape(NS, NCH, 1, CHUNK)
    df = (w * sp * dp).reshape(NS, NCH, 1, CHUNK)
    edata = jnp.stack([gidx, didx], axis=2)
    wdata = jnp.concatenate([a, b, cf, df], axis=2)
    zeros = jnp.zeros((NP, D), jnp.float32)
    x2 = jnp.swapaxes(x.reshape(N, NC, DH), 0, 1).reshape(NC, NP, D)

    p1 = _spmm_sc(x2, edata, wdata, zeros).reshape(NC, N, DH)
    s2 = _mlp_tc(p1, W1, b1, W2)
    p2 = _spmm_sc(s2.reshape(NC, NP, D), edata, wdata, zeros).reshape(NC, N, DH)
    z, emb = _proj_tc(p2, b2, Wp1, bp1, Wp2, bp2)
    return (z, emb)


# speedup vs baseline: 5.6582x; 1.3574x over previous
import functools

import jax
import jax.numpy as jnp
from jax import lax
from jax.experimental import pallas as pl
from jax.experimental.pallas import tpu as pltpu
from jax.experimental.pallas import tpu_sc as plsc

N = 10000
E = 320000
D = 128
DH = 64
NP = N // 2
NC = 2
NS = 16
LANES = 16
EPT = 20480
CHUNK = 80
NCH = EPT // CHUNK
E_PAD = NS * EPT
ROWS_PER_SUB = 312
ROWS_TAIL = NP - NS * ROWS_PER_SUB


def _spmm_sc(table2, edata, wdata, zeros):
    mesh = plsc.VectorSubcoreMesh(core_axis_name="c", subcore_axis_name="s")

    @functools.partial(
        pl.kernel,
        out_type=jax.ShapeDtypeStruct((NC, NP, D), jnp.float32),
        mesh=mesh,
        scratch_types=[
            pltpu.VMEM_SHARED((NP, D), jnp.float32),
            pltpu.VMEM_SHARED((NP, D), jnp.float32),
            pltpu.VMEM((4, CHUNK), jnp.float32),
            pltpu.VMEM((4, CHUNK), jnp.float32),
            pltpu.VMEM((2, CHUNK), jnp.int32),
            pltpu.VMEM((2, CHUNK), jnp.int32),
            pltpu.VMEM((2, CHUNK), jnp.int32),
            pltpu.VMEM((2, CHUNK), jnp.int32),
            pltpu.VMEM((CHUNK, D), jnp.float32),
            pltpu.VMEM((CHUNK, D), jnp.float32),
            pltpu.VMEM((CHUNK, D), jnp.float32),
            pltpu.VMEM((CHUNK, D), jnp.float32),
            pltpu.SemaphoreType.DMA,
            pltpu.SemaphoreType.DMA,
            pltpu.SemaphoreType.DMA,
            pltpu.SemaphoreType.DMA,
            pltpu.SemaphoreType.DMA,
            pltpu.SemaphoreType.DMA,
            pltpu.SemaphoreType.DMA,
            pltpu.SemaphoreType.DMA,
            pltpu.SemaphoreType.DMA,
            pltpu.SemaphoreType.DMA,
        ],
    )
    def spmm_kernel(table_hbm, edata_hbm, w_hbm, zeros_hbm, out_hbm,
                    tbl, acc, wbuf0, wbuf1, ebuf0, ebuf1, ebuf2, ebuf3,
                    rows0, rows1, sbuf0, sbuf1,
                    gsem0, gsem1, esem0, esem1, esem2, esem3,
                    wsem0, wsem1, ssem0, ssem1):
        c = lax.axis_index("c")
        s = lax.axis_index("s")

        rbase = pl.multiple_of(s * ROWS_PER_SUB, 8)
        pltpu.sync_copy(table_hbm.at[c, pl.ds(rbase, ROWS_PER_SUB)],
                        tbl.at[pl.ds(rbase, ROWS_PER_SUB)])
        pltpu.sync_copy(zeros_hbm.at[pl.ds(rbase, ROWS_PER_SUB)],
                        acc.at[pl.ds(rbase, ROWS_PER_SUB)])

        @pl.when(s == NS - 1)
        def _stage_tail():
            tsl = pl.ds(NS * ROWS_PER_SUB, ROWS_TAIL)
            pltpu.sync_copy(table_hbm.at[c, tsl], tbl.at[tsl])
            pltpu.sync_copy(zeros_hbm.at[tsl], acc.at[tsl])

        plsc.subcore_barrier()

        ebufs = (ebuf0, ebuf1, ebuf2, ebuf3)
        esems = (esem0, esem1, esem2, esem3)
        rowss = (rows0, rows1)
        gsems = (gsem0, gsem1)
        wbufs = (wbuf0, wbuf1)
        wsems = (wsem0, wsem1)
        sbufs = (sbuf0, sbuf1)
        ssems = (ssem0, ssem1)

        def process(rows, ebuf, wbuf, sbuf, ssem):
            def group_body(g, _):
                base = g * LANES
                av = wbuf[0, pl.ds(base, LANES)]
                bv = wbuf[1, pl.ds(base, LANES)]
                cv = wbuf[2, pl.ds(base, LANES)]
                dv = wbuf[3, pl.ds(base, LANES)]
                for l in range(LANES):
                    k = base + l
                    a, b, cc, dd = av[l], bv[l], cv[l], dv[l]
                    for jj in range(DH // LANES):
                        slo = pl.ds(jj * LANES, LANES)
                        shi = pl.ds(DH + jj * LANES, LANES)
                        lo = rows[k, slo]
                        hi = rows[k, shi]
                        sbuf[k, slo] = lo * a + hi * b
                        sbuf[k, shi] = lo * cc + hi * dd
                return 0

            lax.fori_loop(0, CHUNK // LANES, group_body, 0)

            pltpu.async_copy(sbuf, acc.at[ebuf.at[1]], ssem, add=True)

        pltpu.async_copy(edata_hbm.at[s, 0], ebuf0, esem0)
        pltpu.async_copy(edata_hbm.at[s, 1], ebuf1, esem1)
        pltpu.async_copy(w_hbm.at[s, 0], wbuf0, wsem0)
        pltpu.async_copy(w_hbm.at[s, 1], wbuf1, wsem1)
        pltpu.make_async_copy(edata_hbm.at[s, 0], ebuf0, esem0).wait()
        pltpu.async_copy(tbl.at[ebuf0.at[0]], rows0, gsem0)

        @pl.loop(0, NCH, step=4)
        def _pipe(jo):
            for b in range(4):
                j = jo + b
                rb = b % 2
                rows, gsem = rowss[rb], gsems[rb]
                wbuf, wsem = wbufs[rb], wsems[rb]
                sbuf, ssem = sbufs[rb], ssems[rb]
                ebuf, esem = ebufs[b], esems[b]
                ebuf_n, esem_n = ebufs[(b + 1) % 4], esems[(b + 1) % 4]
                ebuf_p = ebufs[(b + 2) % 4]

                pltpu.make_async_copy(tbl.at[ebuf.at[0]], rows, gsem).wait()

                @pl.when(j < NCH - 1)
                def _issue_next_gather():
                    pltpu.make_async_copy(
                        edata_hbm.at[s, j + 1], ebuf_n, esem_n).wait()
                    pltpu.async_copy(tbl.at[ebuf_n.at[0]], rowss[1 - rb],
                                     gsems[1 - rb])

                @pl.when(j >= 2)
                def _drain_prev_scatter():
                    pltpu.make_async_copy(
                        sbuf, acc.at[ebuf_p.at[1]], ssem).wait()

                pltpu.make_async_copy(w_hbm.at[s, j], wbuf, wsem).wait()
                process(rows, ebuf, wbuf, sbuf, ssem)

                @pl.when(j < NCH - 2)
                def _issue_next_estage():
                    pltpu.async_copy(edata_hbm.at[s, j + 2], ebuf_p, esems[(b + 2) % 4])
                    pltpu.async_copy(w_hbm.at[s, j + 2], wbuf, wsem)

        pltpu.make_async_copy(sbuf0, acc.at[ebufs[(NCH - 2) % 4].at[1]],
                              ssem0).wait()
        pltpu.make_async_copy(sbuf1, acc.at[ebufs[(NCH - 1) % 4].at[1]],
                              ssem1).wait()

        plsc.subcore_barrier()
        pltpu.sync_copy(acc.at[pl.ds(rbase, ROWS_PER_SUB)],
                        out_hbm.at[c, pl.ds(rbase, ROWS_PER_SUB)])

        @pl.when(s == NS - 1)
        def _write_tail():
            tsl = pl.ds(NS * ROWS_PER_SUB, ROWS_TAIL)
            pltpu.sync_copy(acc.at[tsl], out_hbm.at[c, tsl])

    return spmm_kernel(table2, edata, wdata, zeros)


_RB = 1000

def _mlp_body(p_ref, w1_ref, b1_ref, w2_ref, out_ref):
    agg = jnp.concatenate([p_ref[0], p_ref[1]], axis=-1)
    h = jnp.dot(agg, w1_ref[...], preferred_element_type=jnp.float32,
                precision=lax.Precision.HIGHEST) + b1_ref[...]
    h = jnp.maximum(h, 0.0)
    s2 = jnp.dot(h, w2_ref[...], preferred_element_type=jnp.float32,
                 precision=lax.Precision.HIGHEST)
    out_ref[0] = s2[:, :DH]
    out_ref[1] = s2[:, DH:]


def _mlp_tc(p1, W1, b1, W2):
    grid = N // _RB
    return pl.pallas_call(
        _mlp_body,
        grid=(grid,),
        in_specs=[
            pl.BlockSpec((NC, _RB, DH), lambda i: (0, i, 0)),
            pl.BlockSpec((D, 256), lambda i: (0, 0)),
            pl.BlockSpec((1, 256), lambda i: (0, 0)),
            pl.BlockSpec((256, D), lambda i: (0, 0)),
        ],
        out_specs=pl.BlockSpec((NC, _RB, DH), lambda i: (0, i, 0)),
        out_shape=jax.ShapeDtypeStruct((NC, N, DH), jnp.float32),
    )(p1, W1, b1.reshape(1, 256), W2)


def _proj_body(p_ref, b2_ref, wp1_ref, bp1_ref, wp2_ref, bp2_ref,
               z_ref, emb_ref):
    emb = jnp.concatenate([p_ref[0], p_ref[1]], axis=-1) + b2_ref[...]
    emb_ref[...] = emb
    t = jnp.dot(emb, wp1_ref[...], preferred_element_type=jnp.float32,
                precision=lax.Precision.HIGHEST) + bp1_ref[...]
    t = jnp.maximum(t, 0.0)
    z_ref[...] = jnp.dot(t, wp2_ref[...], preferred_element_type=jnp.float32,
                         precision=lax.Precision.HIGHEST) + bp2_ref[...]


def _proj_tc(p2, b2, Wp1, bp1, Wp2, bp2):
    grid = N // _RB
    return pl.pallas_call(
        _proj_body,
        grid=(grid,),
        in_specs=[
            pl.BlockSpec((NC, _RB, DH), lambda i: (0, i, 0)),
            pl.BlockSpec((1, D), lambda i: (0, 0)),
            pl.BlockSpec((D, D), lambda i: (0, 0)),
            pl.BlockSpec((1, D), lambda i: (0, 0)),
            pl.BlockSpec((D, D), lambda i: (0, 0)),
            pl.BlockSpec((1, D), lambda i: (0, 0)),
        ],
        out_specs=[
            pl.BlockSpec((_RB, D), lambda i: (i, 0)),
            pl.BlockSpec((_RB, D), lambda i: (i, 0)),
        ],
        out_shape=[
            jax.ShapeDtypeStruct((N, D), jnp.float32),
            jax.ShapeDtypeStruct((N, D), jnp.float32),
        ],
    )(p2, b2.reshape(1, D), Wp1, bp1.reshape(1, D), Wp2, bp2.reshape(1, D))


def kernel(x, edge_index, edge_weight, W1, b1, W2, b2, Wp1, bp1, Wp2, bp2):
    pad = E_PAD - E
    src = jnp.concatenate([edge_index[0], jnp.zeros((pad,), jnp.int32)])
    dst = jnp.concatenate([edge_index[1], jnp.zeros((pad,), jnp.int32)])
    w = jnp.concatenate([edge_weight, jnp.zeros((pad,), jnp.float32)])

    gidx = lax.shift_right_logical(src, 1).reshape(NS, NCH, CHUNK)
    didx = lax.shift_right_logical(dst, 1).reshape(NS, NCH, CHUNK)
    sp = (src & 1).astype(jnp.float32)
    dp = (dst & 1).astype(jnp.float32)
    a = (w * (1 - sp) * (1 - dp)).reshape(NS, NCH, 1, CHUNK)
    b = (w * sp * (1 - dp)).reshape(NS, NCH, 1, CHUNK)
    cf = (w * (1 - sp) * dp).reshape(NS, NCH, 1, CHUNK)
    df = (w * sp * dp).reshape(NS, NCH, 1, CHUNK)
    edata = jnp.stack([gidx, didx], axis=2)
    wdata = jnp.concatenate([a, b, cf, df], axis=2)
    zeros = jnp.zeros((NP, D), jnp.float32)
    x2 = jnp.swapaxes(x.reshape(N, NC, DH), 0, 1).reshape(NC, NP, D)

    p1 = _spmm_sc(x2, edata, wdata, zeros).reshape(NC, N, DH)
    s2 = _mlp_tc(p1, W1, b1, W2)
    p2 = _spmm_sc(s2.reshape(NC, NP, D), edata, wdata, zeros).reshape(NC, N, DH)
    z, emb = _proj_tc(p2, b2, Wp1, bp1, Wp2, bp2)
    return (z, emb)


# speedup vs baseline: 6.0919x; 1.0767x over previous
import functools

import jax
import jax.numpy as jnp
from jax import lax
from jax.experimental import pallas as pl
from jax.experimental.pallas import tpu as pltpu
from jax.experimental.pallas import tpu_sc as plsc

N = 10000
E = 320000
D = 128
DH = 64
NP = N // 2
NC = 2
NS = 16
LANES = 16
EPT = 20480
CHUNK = 80
NCH = EPT // CHUNK
E_PAD = NS * EPT
ROWS_PER_SUB = 312
ROWS_TAIL = NP - NS * ROWS_PER_SUB


def _spmm_sc(table2, edata, wdata, zeros):
    mesh = plsc.VectorSubcoreMesh(core_axis_name="c", subcore_axis_name="s")

    @functools.partial(
        pl.kernel,
        out_type=jax.ShapeDtypeStruct((NC, NP, D), jnp.float32),
        mesh=mesh,
        scratch_types=[
            pltpu.VMEM_SHARED((NP, D), jnp.float32),
            pltpu.VMEM_SHARED((NP, D), jnp.float32),
            pltpu.VMEM((4, CHUNK), jnp.float32),
            pltpu.VMEM((4, CHUNK), jnp.float32),
            pltpu.VMEM((2, CHUNK), jnp.int32),
            pltpu.VMEM((2, CHUNK), jnp.int32),
            pltpu.VMEM((2, CHUNK), jnp.int32),
            pltpu.VMEM((2, CHUNK), jnp.int32),
            pltpu.VMEM((CHUNK, D), jnp.float32),
            pltpu.VMEM((CHUNK, D), jnp.float32),
            pltpu.VMEM((CHUNK, D), jnp.float32),
            pltpu.VMEM((CHUNK, D), jnp.float32),
            pltpu.SemaphoreType.DMA,
            pltpu.SemaphoreType.DMA,
            pltpu.SemaphoreType.DMA,
            pltpu.SemaphoreType.DMA,
            pltpu.SemaphoreType.DMA,
            pltpu.SemaphoreType.DMA,
            pltpu.SemaphoreType.DMA,
            pltpu.SemaphoreType.DMA,
            pltpu.SemaphoreType.DMA,
            pltpu.SemaphoreType.DMA,
        ],
    )
    def spmm_kernel(table_hbm, edata_hbm, w_hbm, zeros_hbm, out_hbm,
                    tbl, acc, wbuf0, wbuf1, ebuf0, ebuf1, ebuf2, ebuf3,
                    rows0, rows1, sbuf0, sbuf1,
                    gsem0, gsem1, esem0, esem1, esem2, esem3,
                    wsem0, wsem1, ssem0, ssem1):
        c = lax.axis_index("c")
        s = lax.axis_index("s")

        rbase = pl.multiple_of(s * ROWS_PER_SUB, 8)
        pltpu.sync_copy(table_hbm.at[c, pl.ds(rbase, ROWS_PER_SUB)],
                        tbl.at[pl.ds(rbase, ROWS_PER_SUB)])
        pltpu.sync_copy(zeros_hbm.at[pl.ds(rbase, ROWS_PER_SUB)],
                        acc.at[pl.ds(rbase, ROWS_PER_SUB)])

        @pl.when(s == NS - 1)
        def _stage_tail():
            tsl = pl.ds(NS * ROWS_PER_SUB, ROWS_TAIL)
            pltpu.sync_copy(table_hbm.at[c, tsl], tbl.at[tsl])
            pltpu.sync_copy(zeros_hbm.at[tsl], acc.at[tsl])

        plsc.subcore_barrier()

        ebufs = (ebuf0, ebuf1, ebuf2, ebuf3)
        esems = (esem0, esem1, esem2, esem3)
        rowss = (rows0, rows1)
        gsems = (gsem0, gsem1)
        wbufs = (wbuf0, wbuf1)
        wsems = (wsem0, wsem1)
        sbufs = (sbuf0, sbuf1)
        ssems = (ssem0, ssem1)

        def process(rows, ebuf, wbuf, sbuf, ssem):
            def group_body(g, _):
                base = g * LANES
                av = wbuf[0, pl.ds(base, LANES)]
                bv = wbuf[1, pl.ds(base, LANES)]
                cv = wbuf[2, pl.ds(base, LANES)]
                dv = wbuf[3, pl.ds(base, LANES)]
                for l in range(LANES):
                    k = base + l
                    a, b, cc, dd = av[l], bv[l], cv[l], dv[l]
                    for jj in range(DH // LANES):
                        slo = pl.ds(jj * LANES, LANES)
                        shi = pl.ds(DH + jj * LANES, LANES)
                        lo = rows[k, slo]
                        hi = rows[k, shi]
                        sbuf[k, slo] = lo * a + hi * b
                        sbuf[k, shi] = lo * cc + hi * dd
                return 0

            lax.fori_loop(0, CHUNK // LANES, group_body, 0)

            pltpu.async_copy(sbuf, acc.at[ebuf.at[1]], ssem, add=True)

        pltpu.async_copy(edata_hbm.at[s, 0], ebuf0, esem0)
        pltpu.async_copy(edata_hbm.at[s, 1], ebuf1, esem1)
        pltpu.async_copy(w_hbm.at[s, 0], wbuf0, wsem0)
        pltpu.async_copy(w_hbm.at[s, 1], wbuf1, wsem1)
        pltpu.make_async_copy(edata_hbm.at[s, 0], ebuf0, esem0).wait()
        pltpu.async_copy(tbl.at[ebuf0.at[0]], rows0, gsem0)

        @pl.loop(0, NCH, step=4)
        def _pipe(jo):
            for b in range(4):
                j = jo + b
                rb = b % 2
                rows, gsem = rowss[rb], gsems[rb]
                wbuf, wsem = wbufs[rb], wsems[rb]
                sbuf, ssem = sbufs[rb], ssems[rb]
                ebuf, esem = ebufs[b], esems[b]
                ebuf_n, esem_n = ebufs[(b + 1) % 4], esems[(b + 1) % 4]
                ebuf_p = ebufs[(b + 2) % 4]

                pltpu.make_async_copy(tbl.at[ebuf.at[0]], rows, gsem).wait()

                @pl.when(j < NCH - 1)
                def _issue_next_gather():
                    pltpu.make_async_copy(
                        edata_hbm.at[s, j + 1], ebuf_n, esem_n).wait()
                    pltpu.async_copy(tbl.at[ebuf_n.at[0]], rowss[1 - rb],
                                     gsems[1 - rb])

                @pl.when(j >= 2)
                def _drain_prev_scatter():
                    pltpu.make_async_copy(
                        sbuf, acc.at[ebuf_p.at[1]], ssem).wait()

                pltpu.make_async_copy(w_hbm.at[s, j], wbuf, wsem).wait()
                process(rows, ebuf, wbuf, sbuf, ssem)

                @pl.when(j < NCH - 2)
                def _issue_next_estage():
                    pltpu.async_copy(edata_hbm.at[s, j + 2], ebuf_p, esems[(b + 2) % 4])
                    pltpu.async_copy(w_hbm.at[s, j + 2], wbuf, wsem)

        pltpu.make_async_copy(sbuf0, acc.at[ebufs[(NCH - 2) % 4].at[1]],
                              ssem0).wait()
        pltpu.make_async_copy(sbuf1, acc.at[ebufs[(NCH - 1) % 4].at[1]],
                              ssem1).wait()

        plsc.subcore_barrier()
        pltpu.sync_copy(acc.at[pl.ds(rbase, ROWS_PER_SUB)],
                        out_hbm.at[c, pl.ds(rbase, ROWS_PER_SUB)])

        @pl.when(s == NS - 1)
        def _write_tail():
            tsl = pl.ds(NS * ROWS_PER_SUB, ROWS_TAIL)
            pltpu.sync_copy(acc.at[tsl], out_hbm.at[c, tsl])

    return spmm_kernel(table2, edata, wdata, zeros)


def _prep_body(src_ref, dst_ref, w_ref, e_ref, w4_ref):
    s = src_ref[0, :, 0, :]
    dd = dst_ref[0, :, 0, :]
    w = w_ref[0, :, 0, :]
    e_ref[0, :, 0, :] = lax.shift_right_logical(s, 1)
    e_ref[0, :, 1, :] = lax.shift_right_logical(dd, 1)
    sp = (s & 1).astype(jnp.float32)
    dp = (dd & 1).astype(jnp.float32)
    w4_ref[0, :, 0, :] = w * (1 - sp) * (1 - dp)
    w4_ref[0, :, 1, :] = w * sp * (1 - dp)
    w4_ref[0, :, 2, :] = w * (1 - sp) * dp
    w4_ref[0, :, 3, :] = w * sp * dp


def _prep_tc(srcp, dstp, wp):
    return pl.pallas_call(
        _prep_body,
        grid=(NS,),
        in_specs=[
            pl.BlockSpec((1, NCH, 1, CHUNK), lambda i: (i, 0, 0, 0)),
            pl.BlockSpec((1, NCH, 1, CHUNK), lambda i: (i, 0, 0, 0)),
            pl.BlockSpec((1, NCH, 1, CHUNK), lambda i: (i, 0, 0, 0)),
        ],
        out_specs=[
            pl.BlockSpec((1, NCH, 2, CHUNK), lambda i: (i, 0, 0, 0)),
            pl.BlockSpec((1, NCH, 4, CHUNK), lambda i: (i, 0, 0, 0)),
        ],
        out_shape=[
            jax.ShapeDtypeStruct((NS, NCH, 2, CHUNK), jnp.int32),
            jax.ShapeDtypeStruct((NS, NCH, 4, CHUNK), jnp.float32),
        ],
    )(srcp, dstp, wp)


_RB = 1000

def _mlp_body(p_ref, w1_ref, b1_ref, w2_ref, out_ref):
    agg = jnp.concatenate([p_ref[0], p_ref[1]], axis=-1)
    h = jnp.dot(agg, w1_ref[...], preferred_element_type=jnp.float32) + b1_ref[...]
    h = jnp.maximum(h, 0.0)
    s2 = jnp.dot(h, w2_ref[...], preferred_element_type=jnp.float32)
    out_ref[0] = s2[:, :DH]
    out_ref[1] = s2[:, DH:]


def _mlp_tc(p1, W1, b1, W2):
    grid = N // _RB
    return pl.pallas_call(
        _mlp_body,
        grid=(grid,),
        in_specs=[
            pl.BlockSpec((NC, _RB, DH), lambda i: (0, i, 0)),
            pl.BlockSpec((D, 256), lambda i: (0, 0)),
            pl.BlockSpec((1, 256), lambda i: (0, 0)),
            pl.BlockSpec((256, D), lambda i: (0, 0)),
        ],
        out_specs=pl.BlockSpec((NC, _RB, DH), lambda i: (0, i, 0)),
        out_shape=jax.ShapeDtypeStruct((NC, N, DH), jnp.float32),
    )(p1, W1, b1.reshape(1, 256), W2)


def _proj_body(p_ref, b2_ref, wp1_ref, bp1_ref, wp2_ref, bp2_ref,
               z_ref, emb_ref):
    emb = jnp.concatenate([p_ref[0], p_ref[1]], axis=-1) + b2_ref[...]
    emb_ref[...] = emb
    t = jnp.dot(emb, wp1_ref[...], preferred_element_type=jnp.float32) + bp1_ref[...]
    t = jnp.maximum(t, 0.0)
    z_ref[...] = jnp.dot(t, wp2_ref[...], preferred_element_type=jnp.float32) + bp2_ref[...]


def _proj_tc(p2, b2, Wp1, bp1, Wp2, bp2):
    grid = N // _RB
    return pl.pallas_call(
        _proj_body,
        grid=(grid,),
        in_specs=[
            pl.BlockSpec((NC, _RB, DH), lambda i: (0, i, 0)),
            pl.BlockSpec((1, D), lambda i: (0, 0)),
            pl.BlockSpec((D, D), lambda i: (0, 0)),
            pl.BlockSpec((1, D), lambda i: (0, 0)),
            pl.BlockSpec((D, D), lambda i: (0, 0)),
            pl.BlockSpec((1, D), lambda i: (0, 0)),
        ],
        out_specs=[
            pl.BlockSpec((_RB, D), lambda i: (i, 0)),
            pl.BlockSpec((_RB, D), lambda i: (i, 0)),
        ],
        out_shape=[
            jax.ShapeDtypeStruct((N, D), jnp.float32),
            jax.ShapeDtypeStruct((N, D), jnp.float32),
        ],
    )(p2, b2.reshape(1, D), Wp1, bp1.reshape(1, D), Wp2, bp2.reshape(1, D))


def kernel(x, edge_index, edge_weight, W1, b1, W2, b2, Wp1, bp1, Wp2, bp2):
    pad = E_PAD - E
    src = jnp.concatenate([edge_index[0], jnp.zeros((pad,), jnp.int32)])
    dst = jnp.concatenate([edge_index[1], jnp.zeros((pad,), jnp.int32)])
    w = jnp.concatenate([edge_weight, jnp.zeros((pad,), jnp.float32)])
    edata, wdata = _prep_tc(src.reshape(NS, NCH, 1, CHUNK),
                            dst.reshape(NS, NCH, 1, CHUNK),
                            w.reshape(NS, NCH, 1, CHUNK))
    zeros = jnp.zeros((NP, D), jnp.float32)
    x2 = jnp.swapaxes(x.reshape(N, NC, DH), 0, 1).reshape(NC, NP, D)

    p1 = _spmm_sc(x2, edata, wdata, zeros).reshape(NC, N, DH)
    s2 = _mlp_tc(p1, W1, b1, W2)
    p2 = _spmm_sc(s2.reshape(NC, NP, D), edata, wdata, zeros).reshape(NC, N, DH)
    z, emb = _proj_tc(p2, b2, Wp1, bp1, Wp2, bp2)
    return (z, emb)
